# Initial kernel scaffold; baseline (speedup 1.0000x reference)
import jax
import jax.numpy as jnp
from jax.experimental import pallas as pl


def kernel(x, W_in, b_in, W_g1, b_g1, W_g2, b_g2, ln_gamma, ln_beta, W_out, b_out):
    raise NotImplementedError("write your pallas kernel here")


# speedup vs baseline: 30.8996x; 30.8996x over previous
import jax
import jax.numpy as jnp
from jax.experimental import pallas as pl
from jax.experimental.pallas import tpu as pltpu

BLK = 10000


def _fused_mlp_kernel(x_ref, Win_ref, bin_ref, Wg1_ref, bg1_ref, Wg2_ref,
                      bg2_ref, gam_ref, bet_ref, Wout_ref, bout_ref, out_ref):
    f32 = jnp.float32
    cT = (((1,), (1,)), ((), ()))

    Wc = jax.lax.dot_general(Wg1_ref[...], Win_ref[...],
                             (((1,), (0,)), ((), ())),
                             preferred_element_type=f32)
    bc = jax.lax.dot_general(bin_ref[...], Wg1_ref[...], cT,
                             preferred_element_type=f32) + bg1_ref[...]
    d = float(Wg1_ref.shape[0])
    Wo2 = Wout_ref[...] * (d * gam_ref[...])
    s = jnp.sum(Wout_ref[...] * gam_ref[...], axis=1)[None, :]
    b2 = jax.lax.dot_general(bet_ref[...], Wout_ref[...], cT,
                             preferred_element_type=f32) + bout_ref[...]

    x = x_ref[...]
    h = jax.lax.dot_general(x, Wc, cT, preferred_element_type=f32)
    h = jnp.maximum(h + bc, 0.0)
    h = jax.lax.dot_general(h, Wg2_ref[...], cT, preferred_element_type=f32)
    h = jnp.maximum(h + bg2_ref[...], 0.0)
    s1 = jnp.sum(h, axis=-1, keepdims=True)
    s2 = jnp.sum(h * h, axis=-1, keepdims=True)
    g = jax.lax.rsqrt(d * s2 - (s1 * s1 - d * d * 1e-5))
    p = jax.lax.dot_general(h, Wo2, cT, preferred_element_type=f32)
    out_ref[...] = (p - s1 * s) * g + b2


def kernel(x, W_in, b_in, W_g1, b_g1, W_g2, b_g2, ln_gamma, ln_beta, W_out, b_out):
    n, in_dim = x.shape
    hidden = W_in.shape[0]
    classes = W_out.shape[0]

    grid = (n // BLK,)
    full = lambda shape: pl.BlockSpec(shape, lambda i: (0, 0))
    return pl.pallas_call(
        _fused_mlp_kernel,
        grid=grid,
        in_specs=[
            pl.BlockSpec((BLK, in_dim), lambda i: (i, 0)),
            full((hidden, in_dim)),
            full((1, hidden)),
            full((hidden, hidden)),
            full((1, hidden)),
            full((hidden, hidden)),
            full((1, hidden)),
            full((1, hidden)),
            full((1, hidden)),
            full((classes, hidden)),
            full((1, classes)),
        ],
        out_specs=pl.BlockSpec((BLK, classes), lambda i: (i, 0)),
        out_shape=jax.ShapeDtypeStruct((n, classes), x.dtype),
        compiler_params=pltpu.CompilerParams(
            dimension_semantics=("parallel",)),
    )(x, W_in, b_in.reshape(1, -1), W_g1, b_g1.reshape(1, -1), W_g2,
      b_g2.reshape(1, -1), ln_gamma.reshape(1, -1), ln_beta.reshape(1, -1),
      W_out, b_out.reshape(1, -1))
